# baseline (speedup 1.0000x reference)
import functools
import math

import jax
import jax.numpy as jnp
from jax import lax
from jax.experimental import pallas as pl
from jax.experimental.pallas import tpu as pltpu
from jax.experimental.pallas import tpu_sc as plsc

N = 10000
E = 320000
D = 128
NS = 32
LOG2 = math.log(2.0)
CN = 1.0 / math.sqrt(E / N)

NSC = 2
NSUB = 16
NW = NSC * NSUB
EPW = E // NW
CHUNK = 40
NCHUNK = EPW // CHUNK
GRP = 10
NGRP = NCHUNK // GRP
IRING = 3
NPAD = 10240
RPW = NPAD // NSUB

BN = 1000
BE = 4000


def _pack_words(lo, hi):
    lb = lax.bitcast_convert_type(lo, jnp.int32)
    hb = lax.bitcast_convert_type(hi, jnp.int32)
    lr = lax.shift_right_logical(
        lb + 0x7FFF + (lax.shift_right_logical(lb, 16) & 1), 16)
    hr = (hb + 0x7FFF + (lax.shift_right_logical(hb, 16) & 1)) & jnp.int32(-65536)
    return lr | hr


def _split_cols(w):
    idx = jnp.arange(D // 2)
    base = 32 * (idx // 16) + (idx % 16)
    return w[:, base], w[:, base + 16]


def _node_prep_body(feat_ref, attr_ref, w1_ref, wsc_ref, x_ref, sc_ref):
    f = feat_ref[...]
    x_ref[...] = jnp.dot(f, w1_ref[...], preferred_element_type=jnp.float32)
    sc_ref[...] = f * jnp.dot(attr_ref[...], wsc_ref[...],
                              preferred_element_type=jnp.float32)


def _node_prep(feat, attrs, w1, wsc):
    return pl.pallas_call(
        _node_prep_body,
        grid=(N // BN,),
        in_specs=[
            pl.BlockSpec((BN, D), lambda i: (i, 0)),
            pl.BlockSpec((BN, 16), lambda i: (i, 0)),
            pl.BlockSpec((D, D), lambda i: (0, 0)),
            pl.BlockSpec((16, D), lambda i: (0, 0)),
        ],
        out_specs=[
            pl.BlockSpec((BN, D), lambda i: (i, 0)),
            pl.BlockSpec((BN, D), lambda i: (i, 0)),
        ],
        out_shape=[
            jax.ShapeDtypeStruct((N, D), jnp.float32),
            jax.ShapeDtypeStruct((N, D), jnp.float32),
        ],
    )(feat, attrs, w1, wsc)


def _one_we(ee, ea, wfc1_ref, w2l_ref, w2h_ref, wel_ref, weh_ref):
    h = jnp.dot(ee, wfc1_ref[...], preferred_element_type=jnp.float32)
    h = h * jax.nn.sigmoid(h)
    lo = (jnp.dot(h, w2l_ref[...], preferred_element_type=jnp.float32)
          * jnp.dot(ea, wel_ref[...], preferred_element_type=jnp.float32))
    hi = (jnp.dot(h, w2h_ref[...], preferred_element_type=jnp.float32)
          * jnp.dot(ea, weh_ref[...], preferred_element_type=jnp.float32))
    return _pack_words(lo, hi)


def _fused_a_body(ee_ref, ea_ref,
                  wfc1a_ref, w2la_ref, w2ha_ref, wela_ref, weha_ref,
                  wfc1b_ref, w2lb_ref, w2hb_ref, welb_ref, wehb_ref,
                  feat_ref, attr_ref, w1_ref, wsc_ref,
                  wea_ref, web_ref, x_ref, sc_ref):
    ee = ee_ref[...]
    ea = ea_ref[...]
    wea_ref[...] = _one_we(ee, ea, wfc1a_ref, w2la_ref, w2ha_ref, wela_ref,
                           weha_ref)
    web_ref[...] = _one_we(ee, ea, wfc1b_ref, w2lb_ref, w2hb_ref, welb_ref,
                           wehb_ref)

    @pl.when(pl.program_id(0) < N // BN)
    def _():
        f = feat_ref[...]
        x_ref[...] = jnp.dot(f, w1_ref[...],
                             preferred_element_type=jnp.float32)
        sc_ref[...] = f * jnp.dot(attr_ref[...], wsc_ref[...],
                                  preferred_element_type=jnp.float32)


def _fused_a(ee, ea, wfc1a, wfc2a, wema, wfc1b, wfc2b, wemb, feat, attrs,
             w1, wsc):
    w2la, w2ha = _split_cols(wfc2a)
    wela, weha = _split_cols(wema)
    w2lb, w2hb = _split_cols(wfc2b)
    welb, wehb = _split_cols(wemb)
    nblk = lambda i: (jnp.minimum(i, N // BN - 1), 0)
    return pl.pallas_call(
        _fused_a_body,
        grid=(E // BE,),
        in_specs=[
            pl.BlockSpec((BE, 8), lambda i: (i, 0)),
            pl.BlockSpec((BE, 4), lambda i: (i, 0)),
            pl.BlockSpec((8, 8), lambda i: (0, 0)),
            pl.BlockSpec((8, D // 2), lambda i: (0, 0)),
            pl.BlockSpec((8, D // 2), lambda i: (0, 0)),
            pl.BlockSpec((4, D // 2), lambda i: (0, 0)),
            pl.BlockSpec((4, D // 2), lambda i: (0, 0)),
            pl.BlockSpec((8, 8), lambda i: (0, 0)),
            pl.BlockSpec((8, D // 2), lambda i: (0, 0)),
            pl.BlockSpec((8, D // 2), lambda i: (0, 0)),
            pl.BlockSpec((4, D // 2), lambda i: (0, 0)),
            pl.BlockSpec((4, D // 2), lambda i: (0, 0)),
            pl.BlockSpec((BN, D), nblk),
            pl.BlockSpec((BN, 16), nblk),
            pl.BlockSpec((D, D), lambda i: (0, 0)),
            pl.BlockSpec((16, D), lambda i: (0, 0)),
        ],
        out_specs=[
            pl.BlockSpec((BE, D // 2), lambda i: (i, 0)),
            pl.BlockSpec((BE, D // 2), lambda i: (i, 0)),
            pl.BlockSpec((BN, D), nblk),
            pl.BlockSpec((BN, D), nblk),
        ],
        out_shape=[
            jax.ShapeDtypeStruct((E, D // 2), jnp.int32),
            jax.ShapeDtypeStruct((E, D // 2), jnp.int32),
            jax.ShapeDtypeStruct((N, D), jnp.float32),
            jax.ShapeDtypeStruct((N, D), jnp.float32),
        ],
    )(ee, ea, wfc1a, w2la, w2ha, wela, weha, wfc1b, w2lb, w2hb, welb, wehb,
      feat, attrs, w1, wsc)


def _sc_agg_build():
    mesh = plsc.VectorSubcoreMesh(core_axis_name="c", subcore_axis_name="s")

    @functools.partial(
        pl.kernel,
        mesh=mesh,
        out_type=jax.ShapeDtypeStruct((NSC, NPAD, D), jnp.float32),
        scratch_types=[
            pltpu.VMEM((IRING * GRP, 1, CHUNK), jnp.int32),
            pltpu.VMEM((IRING * GRP, 1, CHUNK), jnp.int32),
            pltpu.VMEM((2, CHUNK, D), jnp.float32),
            pltpu.VMEM((2, CHUNK, D // 2), jnp.int32),
            pltpu.VMEM((2, CHUNK, D), jnp.float32),
            pltpu.VMEM_SHARED((NPAD, D), jnp.float32),
            pltpu.SemaphoreType.DMA,
            pltpu.SemaphoreType.DMA,
            pltpu.SemaphoreType.DMA,
            pltpu.SemaphoreType.DMA,
            pltpu.SemaphoreType.DMA,
            pltpu.SemaphoreType.DMA,
            pltpu.SemaphoreType.DMA,
            pltpu.SemaphoreType.DMA,
        ],
    )
    def sc_agg(x_hbm, we_hbm, src3_hbm, dst3_hbm, zeros_hbm, out_hbm,
               srcv, dstv, rows, webuf, msg, acc_sh,
               sg0, sg1, sw0, sw1, ss0, ss1, six, diy):
        cid = lax.axis_index("c")
        sid = lax.axis_index("s")
        wid = sid * NSC + cid
        r0 = sid * RPW
        sgs, sws, sss = (sg0, sg1), (sw0, sw1), (ss0, ss1)
        pltpu.sync_copy(zeros_hbm, acc_sh.at[pl.ds(r0, RPW)])
        c0 = wid * NCHUNK
        ebase0 = wid * EPW

        def islot(ci):
            return lax.rem(ci // GRP, IRING) * GRP + lax.rem(ci, GRP)

        def idx_issue(g):
            pltpu.async_copy(src3_hbm.at[pl.ds(c0 + g * GRP, GRP)],
                             srcv.at[pl.ds(lax.rem(g, IRING) * GRP, GRP)], six)
            pltpu.async_copy(dst3_hbm.at[pl.ds(c0 + g * GRP, GRP)],
                             dstv.at[pl.ds(lax.rem(g, IRING) * GRP, GRP)], diy)

        def idx_wait(g):
            pltpu.make_async_copy(
                src3_hbm.at[pl.ds(c0 + g * GRP, GRP)],
                srcv.at[pl.ds(lax.rem(g, IRING) * GRP, GRP)], six).wait()
            pltpu.make_async_copy(
                dst3_hbm.at[pl.ds(c0 + g * GRP, GRP)],
                dstv.at[pl.ds(lax.rem(g, IRING) * GRP, GRP)], diy).wait()

        for g0 in range(IRING):
            pltpu.sync_copy(src3_hbm.at[pl.ds(c0 + g0 * GRP, GRP)],
                            srcv.at[pl.ds(g0 * GRP, GRP)])
            pltpu.sync_copy(dst3_hbm.at[pl.ds(c0 + g0 * GRP, GRP)],
                            dstv.at[pl.ds(g0 * GRP, GRP)])
        plsc.subcore_barrier()

        def issue(ci, b):
            pltpu.async_copy(x_hbm.at[srcv.at[islot(ci), 0]], rows.at[b],
                             sgs[b])
            pltpu.async_copy(we_hbm.at[pl.ds(ebase0 + ci * CHUNK, CHUNK)],
                             webuf.at[b], sws[b])

        def step(ci, b, first):
            pltpu.make_async_copy(x_hbm.at[srcv.at[islot(ci), 0]], rows.at[b],
                                  sgs[b]).wait()
            pltpu.make_async_copy(we_hbm.at[pl.ds(ebase0 + ci * CHUNK, CHUNK)],
                                  webuf.at[b], sws[b]).wait()

            @pl.when(jnp.logical_not(first))
            def _():
                pltpu.make_async_copy(msg.at[b],
                                      acc_sh.at[dstv.at[islot(ci), 0]],
                                      sss[b]).wait()

            @plsc.parallel_loop(0, CHUNK, step=1, unroll=4)
            def edge_body(e):
                for j in range(D // 32):
                    wv = webuf[b, e, pl.ds(j * 16, 16)]
                    wlo = lax.bitcast_convert_type(wv << 16, jnp.float32)
                    whi = lax.bitcast_convert_type(wv & -65536, jnp.float32)
                    slo = pl.ds(j * 32, 16)
                    shi = pl.ds(j * 32 + 16, 16)
                    msg[b, e, slo] = rows[b, e, slo] * wlo
                    msg[b, e, shi] = rows[b, e, shi] * whi
            pltpu.async_copy(msg.at[b], acc_sh.at[dstv.at[islot(ci), 0]],
                             sss[b], add=True)

            @pl.when(ci + 2 < NCHUNK)
            def _():
                issue(ci + 2, b)

        issue(0, 0)
        issue(1, 1)

        def group_body(g, carry):
            gb = g * GRP
            step(gb + 0, 0, g == 0)
            step(gb + 1, 1, g == 0)
            @pl.when(jnp.logical_and(g >= 1, g + 2 < NGRP))
            def _():
                idx_issue(g + 2)

            def pair_body(k, c2):
                step(gb + 2 * k, 0, False)
                step(gb + 2 * k + 1, 1, False)
                return c2

            lax.fori_loop(1, 4, pair_body, 0)
            @pl.when(jnp.logical_and(g + 1 >= IRING, g + 1 < NGRP))
            def _():
                idx_wait(g + 1)
            step(gb + 8, 0, False)
            step(gb + 9, 1, False)
            return carry

        lax.fori_loop(0, NGRP, group_body, 0)
        pltpu.make_async_copy(msg.at[0], acc_sh.at[dstv.at[0, 0]],
                              sss[0]).wait()
        pltpu.make_async_copy(msg.at[1], acc_sh.at[dstv.at[1, 0]],
                              sss[1]).wait()
        plsc.subcore_barrier()
        pltpu.sync_copy(acc_sh.at[pl.ds(r0, RPW)],
                        out_hbm.at[cid, pl.ds(r0, RPW)])

    return sc_agg


@functools.cache
def _sc_agg_cached():
    return _sc_agg_build()


def _sc_agg_call(x, we, src, dst, zeros_rpw):
    return _sc_agg_cached()(x, we, src, dst, zeros_rpw)


def _norm_act(p_ref, sc_ref, w2_ref, g_ref, brow_ref):
    agg = (p_ref[0] + p_ref[1]) * CN
    y = jnp.dot(agg, w2_ref[...], preferred_element_type=jnp.float32) + sc_ref[...]
    n2 = jnp.dot(y * y, g_ref[...], preferred_element_type=jnp.float32) + 1e-8
    nrm = jnp.sqrt(n2)
    t = nrm + brow_ref[...]
    sp = jnp.maximum(t, 0.0) + jnp.log1p(jnp.exp(-jnp.abs(t))) - LOG2
    return y * (sp / nrm)


def _finish_body(p_ref, sc_ref, w2_ref, g_ref, brow_ref, out_ref):
    out_ref[...] = _norm_act(p_ref, sc_ref, w2_ref, g_ref, brow_ref)


def _mid_body(p_ref, sc_ref, w2_ref, g_ref, brow_ref, attr_ref, w1_ref,
              wsc_ref, x_ref, sc2_ref):
    h = _norm_act(p_ref, sc_ref, w2_ref, g_ref, brow_ref)
    x_ref[...] = jnp.dot(h, w1_ref[...], preferred_element_type=jnp.float32)
    sc2_ref[...] = h * jnp.dot(attr_ref[...], wsc_ref[...],
                               preferred_element_type=jnp.float32)


def _mid(parts, sc, w2, g, brow, attrs, w1n, wscn):
    return pl.pallas_call(
        _mid_body,
        grid=(N // BN,),
        in_specs=[
            pl.BlockSpec((NSC, BN, D), lambda i: (0, i, 0)),
            pl.BlockSpec((BN, D), lambda i: (i, 0)),
            pl.BlockSpec((D, D), lambda i: (0, 0)),
            pl.BlockSpec((D, D), lambda i: (0, 0)),
            pl.BlockSpec((1, D), lambda i: (0, 0)),
            pl.BlockSpec((BN, 16), lambda i: (i, 0)),
            pl.BlockSpec((D, D), lambda i: (0, 0)),
            pl.BlockSpec((16, D), lambda i: (0, 0)),
        ],
        out_specs=[
            pl.BlockSpec((BN, D), lambda i: (i, 0)),
            pl.BlockSpec((BN, D), lambda i: (i, 0)),
        ],
        out_shape=[
            jax.ShapeDtypeStruct((N, D), jnp.float32),
            jax.ShapeDtypeStruct((N, D), jnp.float32),
        ],
    )(parts, sc, w2, g, brow, attrs, w1n, wscn)


def _finish(parts, sc, w2, g, brow):
    return pl.pallas_call(
        _finish_body,
        grid=(N // BN,),
        in_specs=[
            pl.BlockSpec((NSC, BN, D), lambda i: (0, i, 0)),
            pl.BlockSpec((BN, D), lambda i: (i, 0)),
            pl.BlockSpec((D, D), lambda i: (0, 0)),
            pl.BlockSpec((D, D), lambda i: (0, 0)),
            pl.BlockSpec((1, D), lambda i: (0, 0)),
        ],
        out_specs=pl.BlockSpec((BN, D), lambda i: (i, 0)),
        out_shape=jax.ShapeDtypeStruct((N, D), jnp.float32),
    )(parts, sc, w2, g, brow)


def _norm_groups():
    p = jnp.arange(D)[:, None]
    q = jnp.arange(D)[None, :]
    scal = (p == q) & (q < NS)
    vec = (p >= NS) & (q >= NS) & ((p - NS) // 3 == (q - NS) // 3)
    return (scal | vec).astype(jnp.float32)


def kernel(node_features, node_attrs, edge_index, edge_embedding, edge_attrs,
           W1_0, Wfc1_0, Wfc2_0, We_0, W2_0, Wsc_0, bias_0,
           W1_1, Wfc1_1, Wfc2_1, We_1, W2_1, Wsc_1, bias_1):
    src = edge_index[0].astype(jnp.int32).reshape(E // CHUNK, 1, CHUNK)
    dst = edge_index[1].astype(jnp.int32).reshape(E // CHUNK, 1, CHUNK)
    zeros_rpw = jnp.zeros((RPW, D), dtype=jnp.float32)
    g = _norm_groups()
    brow0 = jnp.where(jnp.arange(D) < NS, bias_0[0], bias_0[1])[None, :]
    brow1 = jnp.where(jnp.arange(D) < NS, bias_1[0], bias_1[1])[None, :]

    we1, we2, x1, sc1 = _fused_a(edge_embedding, edge_attrs,
                                 Wfc1_0, Wfc2_0, We_0,
                                 Wfc1_1, Wfc2_1, We_1,
                                 node_features, node_attrs, W1_0, Wsc_0)
    parts1 = _sc_agg_call(x1, we1, src, dst, zeros_rpw)
    x2, sc2 = _mid(parts1, sc1, W2_0, g, brow0, node_attrs, W1_1, Wsc_1)
    parts2 = _sc_agg_call(x2, we2, src, dst, zeros_rpw)
    return _finish(parts2, sc2, W2_1, g, brow1)


# speedup vs baseline: 1.0905x; 1.0905x over previous
import functools
import math

import jax
import jax.numpy as jnp
from jax import lax
from jax.experimental import pallas as pl
from jax.experimental.pallas import tpu as pltpu
from jax.experimental.pallas import tpu_sc as plsc

N = 10000
E = 320000
D = 128
NS = 32
LOG2 = math.log(2.0)
CN = 1.0 / math.sqrt(E / N)

NSC = 2
NSUB = 16
NW = NSC * NSUB
EPW = E // NW
CHUNK = 80
NCHUNK = EPW // CHUNK
GRP = 5
NGRP = NCHUNK // GRP
IRING = 3
NPAD = 10240
RPW = NPAD // NSUB

BN = 1000
BE = 4000


def _pack_words(lo, hi):
    lb = lax.bitcast_convert_type(lo, jnp.int32)
    hb = lax.bitcast_convert_type(hi, jnp.int32)
    lr = lax.shift_right_logical(
        lb + 0x7FFF + (lax.shift_right_logical(lb, 16) & 1), 16)
    hr = (hb + 0x7FFF + (lax.shift_right_logical(hb, 16) & 1)) & jnp.int32(-65536)
    return lr | hr


def _split_cols(w):
    idx = jnp.arange(D // 2)
    base = 32 * (idx // 16) + (idx % 16)
    return w[:, base], w[:, base + 16]


def _node_prep_body(feat_ref, attr_ref, w1_ref, wsc_ref, x_ref, sc_ref):
    f = feat_ref[...]
    x_ref[...] = jnp.dot(f, w1_ref[...], preferred_element_type=jnp.float32)
    sc_ref[...] = f * jnp.dot(attr_ref[...], wsc_ref[...],
                              preferred_element_type=jnp.float32)


def _node_prep(feat, attrs, w1, wsc):
    return pl.pallas_call(
        _node_prep_body,
        grid=(N // BN,),
        in_specs=[
            pl.BlockSpec((BN, D), lambda i: (i, 0)),
            pl.BlockSpec((BN, 16), lambda i: (i, 0)),
            pl.BlockSpec((D, D), lambda i: (0, 0)),
            pl.BlockSpec((16, D), lambda i: (0, 0)),
        ],
        out_specs=[
            pl.BlockSpec((BN, D), lambda i: (i, 0)),
            pl.BlockSpec((BN, D), lambda i: (i, 0)),
        ],
        out_shape=[
            jax.ShapeDtypeStruct((N, D), jnp.float32),
            jax.ShapeDtypeStruct((N, D), jnp.float32),
        ],
    )(feat, attrs, w1, wsc)


def _one_we(ee, ea, wfc1_ref, w2l_ref, w2h_ref, wel_ref, weh_ref):
    h = jnp.dot(ee, wfc1_ref[...], preferred_element_type=jnp.float32)
    h = h * jax.nn.sigmoid(h)
    lo = (jnp.dot(h, w2l_ref[...], preferred_element_type=jnp.float32)
          * jnp.dot(ea, wel_ref[...], preferred_element_type=jnp.float32))
    hi = (jnp.dot(h, w2h_ref[...], preferred_element_type=jnp.float32)
          * jnp.dot(ea, weh_ref[...], preferred_element_type=jnp.float32))
    return _pack_words(lo, hi)


def _fused_a_body(ee_ref, ea_ref,
                  wfc1a_ref, w2la_ref, w2ha_ref, wela_ref, weha_ref,
                  wfc1b_ref, w2lb_ref, w2hb_ref, welb_ref, wehb_ref,
                  feat_ref, attr_ref, w1_ref, wsc_ref,
                  wea_ref, web_ref, x_ref, sc_ref):
    ee = ee_ref[...]
    ea = ea_ref[...]
    wea_ref[...] = _one_we(ee, ea, wfc1a_ref, w2la_ref, w2ha_ref, wela_ref,
                           weha_ref)
    web_ref[...] = _one_we(ee, ea, wfc1b_ref, w2lb_ref, w2hb_ref, welb_ref,
                           wehb_ref)

    @pl.when(pl.program_id(0) < N // BN)
    def _():
        f = feat_ref[...]
        x_ref[...] = jnp.dot(f, w1_ref[...],
                             preferred_element_type=jnp.float32)
        sc_ref[...] = f * jnp.dot(attr_ref[...], wsc_ref[...],
                                  preferred_element_type=jnp.float32)


def _fused_a(ee, ea, wfc1a, wfc2a, wema, wfc1b, wfc2b, wemb, feat, attrs,
             w1, wsc):
    w2la, w2ha = _split_cols(wfc2a)
    wela, weha = _split_cols(wema)
    w2lb, w2hb = _split_cols(wfc2b)
    welb, wehb = _split_cols(wemb)
    nblk = lambda i: (jnp.minimum(i, N // BN - 1), 0)
    return pl.pallas_call(
        _fused_a_body,
        grid=(E // BE,),
        in_specs=[
            pl.BlockSpec((BE, 8), lambda i: (i, 0)),
            pl.BlockSpec((BE, 4), lambda i: (i, 0)),
            pl.BlockSpec((8, 8), lambda i: (0, 0)),
            pl.BlockSpec((8, D // 2), lambda i: (0, 0)),
            pl.BlockSpec((8, D // 2), lambda i: (0, 0)),
            pl.BlockSpec((4, D // 2), lambda i: (0, 0)),
            pl.BlockSpec((4, D // 2), lambda i: (0, 0)),
            pl.BlockSpec((8, 8), lambda i: (0, 0)),
            pl.BlockSpec((8, D // 2), lambda i: (0, 0)),
            pl.BlockSpec((8, D // 2), lambda i: (0, 0)),
            pl.BlockSpec((4, D // 2), lambda i: (0, 0)),
            pl.BlockSpec((4, D // 2), lambda i: (0, 0)),
            pl.BlockSpec((BN, D), nblk),
            pl.BlockSpec((BN, 16), nblk),
            pl.BlockSpec((D, D), lambda i: (0, 0)),
            pl.BlockSpec((16, D), lambda i: (0, 0)),
        ],
        out_specs=[
            pl.BlockSpec((BE, D // 2), lambda i: (i, 0)),
            pl.BlockSpec((BE, D // 2), lambda i: (i, 0)),
            pl.BlockSpec((BN, D), nblk),
            pl.BlockSpec((BN, D), nblk),
        ],
        out_shape=[
            jax.ShapeDtypeStruct((E, D // 2), jnp.int32),
            jax.ShapeDtypeStruct((E, D // 2), jnp.int32),
            jax.ShapeDtypeStruct((N, D), jnp.float32),
            jax.ShapeDtypeStruct((N, D), jnp.float32),
        ],
    )(ee, ea, wfc1a, w2la, w2ha, wela, weha, wfc1b, w2lb, w2hb, welb, wehb,
      feat, attrs, w1, wsc)


def _sc_agg_build():
    mesh = plsc.VectorSubcoreMesh(core_axis_name="c", subcore_axis_name="s")

    @functools.partial(
        pl.kernel,
        mesh=mesh,
        out_type=jax.ShapeDtypeStruct((NSC, NPAD, D), jnp.float32),
        scratch_types=[
            pltpu.VMEM((IRING * GRP, 1, CHUNK), jnp.int32),
            pltpu.VMEM((IRING * GRP, 1, CHUNK), jnp.int32),
            pltpu.VMEM((2, CHUNK, D), jnp.float32),
            pltpu.VMEM((2, CHUNK, D // 2), jnp.int32),
            pltpu.VMEM_SHARED((NPAD, D), jnp.float32),
            pltpu.SemaphoreType.DMA,
            pltpu.SemaphoreType.DMA,
            pltpu.SemaphoreType.DMA,
            pltpu.SemaphoreType.DMA,
            pltpu.SemaphoreType.DMA,
            pltpu.SemaphoreType.DMA,
            pltpu.SemaphoreType.DMA,
            pltpu.SemaphoreType.DMA,
        ],
    )
    def sc_agg(x_hbm, we_hbm, src3_hbm, dst3_hbm, zeros_hbm, out_hbm,
               srcv, dstv, rows, webuf, acc_sh,
               sg0, sg1, sw0, sw1, ss0, ss1, six, diy):
        cid = lax.axis_index("c")
        sid = lax.axis_index("s")
        wid = sid * NSC + cid
        r0 = sid * RPW
        sgs, sws, sss = (sg0, sg1), (sw0, sw1), (ss0, ss1)
        pltpu.sync_copy(zeros_hbm, acc_sh.at[pl.ds(r0, RPW)])
        c0 = wid * NCHUNK
        ebase0 = wid * EPW

        def islot(ci):
            return lax.rem(ci // GRP, IRING) * GRP + lax.rem(ci, GRP)

        def idx_issue(g):
            pltpu.async_copy(src3_hbm.at[pl.ds(c0 + g * GRP, GRP)],
                             srcv.at[pl.ds(lax.rem(g, IRING) * GRP, GRP)], six)
            pltpu.async_copy(dst3_hbm.at[pl.ds(c0 + g * GRP, GRP)],
                             dstv.at[pl.ds(lax.rem(g, IRING) * GRP, GRP)], diy)

        def idx_wait(g):
            pltpu.make_async_copy(
                src3_hbm.at[pl.ds(c0 + g * GRP, GRP)],
                srcv.at[pl.ds(lax.rem(g, IRING) * GRP, GRP)], six).wait()
            pltpu.make_async_copy(
                dst3_hbm.at[pl.ds(c0 + g * GRP, GRP)],
                dstv.at[pl.ds(lax.rem(g, IRING) * GRP, GRP)], diy).wait()

        for g0 in range(IRING):
            pltpu.sync_copy(src3_hbm.at[pl.ds(c0 + g0 * GRP, GRP)],
                            srcv.at[pl.ds(g0 * GRP, GRP)])
            pltpu.sync_copy(dst3_hbm.at[pl.ds(c0 + g0 * GRP, GRP)],
                            dstv.at[pl.ds(g0 * GRP, GRP)])
        plsc.subcore_barrier()

        def issue(ci, b):
            pltpu.async_copy(x_hbm.at[srcv.at[islot(ci), 0]], rows.at[b],
                             sgs[b])
            pltpu.async_copy(we_hbm.at[pl.ds(ebase0 + ci * CHUNK, CHUNK)],
                             webuf.at[b], sws[b])

        def step(ci, b):
            g = ci // GRP
            pltpu.make_async_copy(x_hbm.at[srcv.at[islot(ci), 0]], rows.at[b],
                                  sgs[b]).wait()
            pltpu.make_async_copy(we_hbm.at[pl.ds(ebase0 + ci * CHUNK, CHUNK)],
                                  webuf.at[b], sws[b]).wait()

            @pl.when(jnp.logical_and(lax.rem(ci, GRP) == 1,
                                     jnp.logical_and(g + 2 >= IRING,
                                                     g + 2 < NGRP)))
            def _():
                idx_issue(g + 2)

            @pl.when(jnp.logical_and(lax.rem(ci, GRP) == 3,
                                     jnp.logical_and(g + 1 >= IRING,
                                                     g + 1 < NGRP)))
            def _():
                idx_wait(g + 1)

            @plsc.parallel_loop(0, CHUNK, step=1, unroll=4)
            def edge_body(e):
                for j in range(D // 32):
                    wv = webuf[b, e, pl.ds(j * 16, 16)]
                    wlo = lax.bitcast_convert_type(wv << 16, jnp.float32)
                    whi = lax.bitcast_convert_type(wv & -65536, jnp.float32)
                    slo = pl.ds(j * 32, 16)
                    shi = pl.ds(j * 32 + 16, 16)
                    rows[b, e, slo] = rows[b, e, slo] * wlo
                    rows[b, e, shi] = rows[b, e, shi] * whi

            pltpu.async_copy(rows.at[b], acc_sh.at[dstv.at[islot(ci), 0]],
                             sss[b], add=True)

            @pl.when(ci + 2 < NCHUNK)
            def _():
                pltpu.async_copy(we_hbm.at[pl.ds(ebase0 + (ci + 2) * CHUNK,
                                                 CHUNK)],
                                 webuf.at[b], sws[b])
            pltpu.make_async_copy(rows.at[b], acc_sh.at[dstv.at[islot(ci), 0]],
                                  sss[b]).wait()

            @pl.when(ci + 2 < NCHUNK)
            def _():
                pltpu.async_copy(x_hbm.at[srcv.at[islot(ci + 2), 0]],
                                 rows.at[b], sgs[b])

        issue(0, 0)
        issue(1, 1)

        def pair_body(k, carry):
            step(2 * k, 0)
            step(2 * k + 1, 1)
            return carry

        lax.fori_loop(0, NCHUNK // 2, pair_body, 0)
        step(NCHUNK - 1, 0)
        plsc.subcore_barrier()
        pltpu.sync_copy(acc_sh.at[pl.ds(r0, RPW)],
                        out_hbm.at[cid, pl.ds(r0, RPW)])

    return sc_agg


@functools.cache
def _sc_agg_cached():
    return _sc_agg_build()


def _sc_agg_call(x, we, src, dst, zeros_rpw):
    return _sc_agg_cached()(x, we, src, dst, zeros_rpw)


def _norm_act(p_ref, sc_ref, w2_ref, g_ref, brow_ref):
    agg = (p_ref[0] + p_ref[1]) * CN
    y = jnp.dot(agg, w2_ref[...], preferred_element_type=jnp.float32) + sc_ref[...]
    n2 = jnp.dot(y * y, g_ref[...], preferred_element_type=jnp.float32) + 1e-8
    nrm = jnp.sqrt(n2)
    t = nrm + brow_ref[...]
    sp = jnp.maximum(t, 0.0) + jnp.log1p(jnp.exp(-jnp.abs(t))) - LOG2
    return y * (sp / nrm)


def _finish_body(p_ref, sc_ref, w2_ref, g_ref, brow_ref, out_ref):
    out_ref[...] = _norm_act(p_ref, sc_ref, w2_ref, g_ref, brow_ref)


def _mid_body(p_ref, sc_ref, w2_ref, g_ref, brow_ref, attr_ref, w1_ref,
              wsc_ref, x_ref, sc2_ref):
    h = _norm_act(p_ref, sc_ref, w2_ref, g_ref, brow_ref)
    x_ref[...] = jnp.dot(h, w1_ref[...], preferred_element_type=jnp.float32)
    sc2_ref[...] = h * jnp.dot(attr_ref[...], wsc_ref[...],
                               preferred_element_type=jnp.float32)


def _mid(parts, sc, w2, g, brow, attrs, w1n, wscn):
    return pl.pallas_call(
        _mid_body,
        grid=(N // BN,),
        in_specs=[
            pl.BlockSpec((NSC, BN, D), lambda i: (0, i, 0)),
            pl.BlockSpec((BN, D), lambda i: (i, 0)),
            pl.BlockSpec((D, D), lambda i: (0, 0)),
            pl.BlockSpec((D, D), lambda i: (0, 0)),
            pl.BlockSpec((1, D), lambda i: (0, 0)),
            pl.BlockSpec((BN, 16), lambda i: (i, 0)),
            pl.BlockSpec((D, D), lambda i: (0, 0)),
            pl.BlockSpec((16, D), lambda i: (0, 0)),
        ],
        out_specs=[
            pl.BlockSpec((BN, D), lambda i: (i, 0)),
            pl.BlockSpec((BN, D), lambda i: (i, 0)),
        ],
        out_shape=[
            jax.ShapeDtypeStruct((N, D), jnp.float32),
            jax.ShapeDtypeStruct((N, D), jnp.float32),
        ],
    )(parts, sc, w2, g, brow, attrs, w1n, wscn)


def _finish(parts, sc, w2, g, brow):
    return pl.pallas_call(
        _finish_body,
        grid=(N // BN,),
        in_specs=[
            pl.BlockSpec((NSC, BN, D), lambda i: (0, i, 0)),
            pl.BlockSpec((BN, D), lambda i: (i, 0)),
            pl.BlockSpec((D, D), lambda i: (0, 0)),
            pl.BlockSpec((D, D), lambda i: (0, 0)),
            pl.BlockSpec((1, D), lambda i: (0, 0)),
        ],
        out_specs=pl.BlockSpec((BN, D), lambda i: (i, 0)),
        out_shape=jax.ShapeDtypeStruct((N, D), jnp.float32),
    )(parts, sc, w2, g, brow)


def _norm_groups():
    p = jnp.arange(D)[:, None]
    q = jnp.arange(D)[None, :]
    scal = (p == q) & (q < NS)
    vec = (p >= NS) & (q >= NS) & ((p - NS) // 3 == (q - NS) // 3)
    return (scal | vec).astype(jnp.float32)


def kernel(node_features, node_attrs, edge_index, edge_embedding, edge_attrs,
           W1_0, Wfc1_0, Wfc2_0, We_0, W2_0, Wsc_0, bias_0,
           W1_1, Wfc1_1, Wfc2_1, We_1, W2_1, Wsc_1, bias_1):
    src = edge_index[0].astype(jnp.int32).reshape(E // CHUNK, 1, CHUNK)
    dst = edge_index[1].astype(jnp.int32).reshape(E // CHUNK, 1, CHUNK)
    zeros_rpw = jnp.zeros((RPW, D), dtype=jnp.float32)
    g = _norm_groups()
    brow0 = jnp.where(jnp.arange(D) < NS, bias_0[0], bias_0[1])[None, :]
    brow1 = jnp.where(jnp.arange(D) < NS, bias_1[0], bias_1[1])[None, :]

    we1, we2, x1, sc1 = _fused_a(edge_embedding, edge_attrs,
                                 Wfc1_0, Wfc2_0, We_0,
                                 Wfc1_1, Wfc2_1, We_1,
                                 node_features, node_attrs, W1_0, Wsc_0)
    parts1 = _sc_agg_call(x1, we1, src, dst, zeros_rpw)
    x2, sc2 = _mid(parts1, sc1, W2_0, g, brow0, node_attrs, W1_1, Wsc_1)
    parts2 = _sc_agg_call(x2, we2, src, dst, zeros_rpw)
    return _finish(parts2, sc2, W2_1, g, brow1)


# speedup vs baseline: 1.1069x; 1.0150x over previous
import functools
import math

import jax
import jax.numpy as jnp
from jax import lax
from jax.experimental import pallas as pl
from jax.experimental.pallas import tpu as pltpu
from jax.experimental.pallas import tpu_sc as plsc

N = 10000
E = 320000
D = 128
NS = 32
LOG2 = math.log(2.0)
CN = 1.0 / math.sqrt(E / N)

NSC = 2
NSUB = 16
NW = NSC * NSUB
EPW = E // NW
CHUNK = 80
NCHUNK = EPW // CHUNK
GRP = 5
NGRP = NCHUNK // GRP
IRING = 3
NPAD = 10240
RPW = NPAD // NSUB

BN = 1000
BE = 4000


def _pack_words(lo, hi):
    lo16 = lax.bitcast_convert_type(lo.astype(jnp.bfloat16), jnp.uint16)
    hi16 = lax.bitcast_convert_type(hi.astype(jnp.bfloat16), jnp.uint16)
    return (hi16.astype(jnp.int32) << 16) | lo16.astype(jnp.int32)


def _split_cols(w):
    idx = jnp.arange(D // 2)
    base = 32 * (idx // 16) + (idx % 16)
    return w[:, base], w[:, base + 16]


def _node_prep_body(feat_ref, attr_ref, w1_ref, wsc_ref, x_ref, sc_ref):
    f = feat_ref[...]
    x_ref[...] = jnp.dot(f, w1_ref[...], preferred_element_type=jnp.float32)
    sc_ref[...] = f * jnp.dot(attr_ref[...], wsc_ref[...],
                              preferred_element_type=jnp.float32)


def _node_prep(feat, attrs, w1, wsc):
    return pl.pallas_call(
        _node_prep_body,
        grid=(N // BN,),
        in_specs=[
            pl.BlockSpec((BN, D), lambda i: (i, 0)),
            pl.BlockSpec((BN, 16), lambda i: (i, 0)),
            pl.BlockSpec((D, D), lambda i: (0, 0)),
            pl.BlockSpec((16, D), lambda i: (0, 0)),
        ],
        out_specs=[
            pl.BlockSpec((BN, D), lambda i: (i, 0)),
            pl.BlockSpec((BN, D), lambda i: (i, 0)),
        ],
        out_shape=[
            jax.ShapeDtypeStruct((N, D), jnp.float32),
            jax.ShapeDtypeStruct((N, D), jnp.float32),
        ],
    )(feat, attrs, w1, wsc)


def _one_we(ee, ea, wfc1_ref, w2l_ref, w2h_ref, wel_ref, weh_ref):
    h = jnp.dot(ee, wfc1_ref[...], preferred_element_type=jnp.float32)
    h = h * jax.nn.sigmoid(h)
    lo = (jnp.dot(h, w2l_ref[...], preferred_element_type=jnp.float32)
          * jnp.dot(ea, wel_ref[...], preferred_element_type=jnp.float32))
    hi = (jnp.dot(h, w2h_ref[...], preferred_element_type=jnp.float32)
          * jnp.dot(ea, weh_ref[...], preferred_element_type=jnp.float32))
    return _pack_words(lo, hi)


def _fused_a_body(ee_ref, ea_ref,
                  wfc1a_ref, w2la_ref, w2ha_ref, wela_ref, weha_ref,
                  wfc1b_ref, w2lb_ref, w2hb_ref, welb_ref, wehb_ref,
                  feat_ref, attr_ref, w1_ref, wsc_ref,
                  wea_ref, web_ref, x_ref, sc_ref):
    ee = ee_ref[...]
    ea = ea_ref[...]
    wea_ref[...] = _one_we(ee, ea, wfc1a_ref, w2la_ref, w2ha_ref, wela_ref,
                           weha_ref)
    web_ref[...] = _one_we(ee, ea, wfc1b_ref, w2lb_ref, w2hb_ref, welb_ref,
                           wehb_ref)

    @pl.when(pl.program_id(0) < N // BN)
    def _():
        f = feat_ref[...]
        x_ref[...] = jnp.dot(f, w1_ref[...],
                             preferred_element_type=jnp.float32)
        sc_ref[...] = f * jnp.dot(attr_ref[...], wsc_ref[...],
                                  preferred_element_type=jnp.float32)


def _fused_a(ee, ea, wfc1a, wfc2a, wema, wfc1b, wfc2b, wemb, feat, attrs,
             w1, wsc):
    w2la, w2ha = _split_cols(wfc2a)
    wela, weha = _split_cols(wema)
    w2lb, w2hb = _split_cols(wfc2b)
    welb, wehb = _split_cols(wemb)
    nblk = lambda i: (jnp.minimum(i, N // BN - 1), 0)
    return pl.pallas_call(
        _fused_a_body,
        grid=(E // BE,),
        in_specs=[
            pl.BlockSpec((BE, 8), lambda i: (i, 0)),
            pl.BlockSpec((BE, 4), lambda i: (i, 0)),
            pl.BlockSpec((8, 8), lambda i: (0, 0)),
            pl.BlockSpec((8, D // 2), lambda i: (0, 0)),
            pl.BlockSpec((8, D // 2), lambda i: (0, 0)),
            pl.BlockSpec((4, D // 2), lambda i: (0, 0)),
            pl.BlockSpec((4, D // 2), lambda i: (0, 0)),
            pl.BlockSpec((8, 8), lambda i: (0, 0)),
            pl.BlockSpec((8, D // 2), lambda i: (0, 0)),
            pl.BlockSpec((8, D // 2), lambda i: (0, 0)),
            pl.BlockSpec((4, D // 2), lambda i: (0, 0)),
            pl.BlockSpec((4, D // 2), lambda i: (0, 0)),
            pl.BlockSpec((BN, D), nblk),
            pl.BlockSpec((BN, 16), nblk),
            pl.BlockSpec((D, D), lambda i: (0, 0)),
            pl.BlockSpec((16, D), lambda i: (0, 0)),
        ],
        out_specs=[
            pl.BlockSpec((BE, D // 2), lambda i: (i, 0)),
            pl.BlockSpec((BE, D // 2), lambda i: (i, 0)),
            pl.BlockSpec((BN, D), nblk),
            pl.BlockSpec((BN, D), nblk),
        ],
        out_shape=[
            jax.ShapeDtypeStruct((E, D // 2), jnp.int32),
            jax.ShapeDtypeStruct((E, D // 2), jnp.int32),
            jax.ShapeDtypeStruct((N, D), jnp.float32),
            jax.ShapeDtypeStruct((N, D), jnp.float32),
        ],
    )(ee, ea, wfc1a, w2la, w2ha, wela, weha, wfc1b, w2lb, w2hb, welb, wehb,
      feat, attrs, w1, wsc)


def _sc_agg_build():
    mesh = plsc.VectorSubcoreMesh(core_axis_name="c", subcore_axis_name="s")

    @functools.partial(
        pl.kernel,
        mesh=mesh,
        out_type=jax.ShapeDtypeStruct((NSC, NPAD, D), jnp.float32),
        scratch_types=[
            pltpu.VMEM((IRING * GRP, 1, CHUNK), jnp.int32),
            pltpu.VMEM((IRING * GRP, 1, CHUNK), jnp.int32),
            pltpu.VMEM((2, CHUNK, D), jnp.float32),
            pltpu.VMEM((2, CHUNK, D // 2), jnp.int32),
            pltpu.VMEM_SHARED((NPAD, D), jnp.float32),
            pltpu.SemaphoreType.DMA,
            pltpu.SemaphoreType.DMA,
            pltpu.SemaphoreType.DMA,
            pltpu.SemaphoreType.DMA,
            pltpu.SemaphoreType.DMA,
            pltpu.SemaphoreType.DMA,
            pltpu.SemaphoreType.DMA,
            pltpu.SemaphoreType.DMA,
        ],
    )
    def sc_agg(x_hbm, we_hbm, src3_hbm, dst3_hbm, zeros_hbm, out_hbm,
               srcv, dstv, rows, webuf, acc_sh,
               sg0, sg1, sw0, sw1, ss0, ss1, six, diy):
        cid = lax.axis_index("c")
        sid = lax.axis_index("s")
        wid = sid * NSC + cid
        r0 = sid * RPW
        sgs, sws, sss = (sg0, sg1), (sw0, sw1), (ss0, ss1)
        pltpu.sync_copy(zeros_hbm, acc_sh.at[pl.ds(r0, RPW)])
        c0 = wid * NCHUNK
        ebase0 = wid * EPW

        def islot(ci):
            return lax.rem(ci // GRP, IRING) * GRP + lax.rem(ci, GRP)

        def idx_issue(g):
            pltpu.async_copy(src3_hbm.at[pl.ds(c0 + g * GRP, GRP)],
                             srcv.at[pl.ds(lax.rem(g, IRING) * GRP, GRP)], six)
            pltpu.async_copy(dst3_hbm.at[pl.ds(c0 + g * GRP, GRP)],
                             dstv.at[pl.ds(lax.rem(g, IRING) * GRP, GRP)], diy)

        def idx_wait(g):
            pltpu.make_async_copy(
                src3_hbm.at[pl.ds(c0 + g * GRP, GRP)],
                srcv.at[pl.ds(lax.rem(g, IRING) * GRP, GRP)], six).wait()
            pltpu.make_async_copy(
                dst3_hbm.at[pl.ds(c0 + g * GRP, GRP)],
                dstv.at[pl.ds(lax.rem(g, IRING) * GRP, GRP)], diy).wait()

        for g0 in range(IRING):
            pltpu.sync_copy(src3_hbm.at[pl.ds(c0 + g0 * GRP, GRP)],
                            srcv.at[pl.ds(g0 * GRP, GRP)])
            pltpu.sync_copy(dst3_hbm.at[pl.ds(c0 + g0 * GRP, GRP)],
                            dstv.at[pl.ds(g0 * GRP, GRP)])
        plsc.subcore_barrier()

        def issue(ci, b):
            pltpu.async_copy(x_hbm.at[srcv.at[islot(ci), 0]], rows.at[b],
                             sgs[b])
            pltpu.async_copy(we_hbm.at[pl.ds(ebase0 + ci * CHUNK, CHUNK)],
                             webuf.at[b], sws[b])

        def step(ci, b):
            g = ci // GRP
            pltpu.make_async_copy(x_hbm.at[srcv.at[islot(ci), 0]], rows.at[b],
                                  sgs[b]).wait()
            pltpu.make_async_copy(we_hbm.at[pl.ds(ebase0 + ci * CHUNK, CHUNK)],
                                  webuf.at[b], sws[b]).wait()

            @pl.when(jnp.logical_and(lax.rem(ci, GRP) == 1,
                                     jnp.logical_and(g + 2 >= IRING,
                                                     g + 2 < NGRP)))
            def _():
                idx_issue(g + 2)

            @pl.when(jnp.logical_and(lax.rem(ci, GRP) == 3,
                                     jnp.logical_and(g + 1 >= IRING,
                                                     g + 1 < NGRP)))
            def _():
                idx_wait(g + 1)

            @plsc.parallel_loop(0, CHUNK, step=1, unroll=4)
            def edge_body(e):
                for j in range(D // 32):
                    wv = webuf[b, e, pl.ds(j * 16, 16)]
                    wlo = lax.bitcast_convert_type(wv << 16, jnp.float32)
                    whi = lax.bitcast_convert_type(wv & -65536, jnp.float32)
                    slo = pl.ds(j * 32, 16)
                    shi = pl.ds(j * 32 + 16, 16)
                    rows[b, e, slo] = rows[b, e, slo] * wlo
                    rows[b, e, shi] = rows[b, e, shi] * whi

            pltpu.async_copy(rows.at[b], acc_sh.at[dstv.at[islot(ci), 0]],
                             sss[b], add=True)

            @pl.when(ci + 2 < NCHUNK)
            def _():
                pltpu.async_copy(we_hbm.at[pl.ds(ebase0 + (ci + 2) * CHUNK,
                                                 CHUNK)],
                                 webuf.at[b], sws[b])
            pltpu.make_async_copy(rows.at[b], acc_sh.at[dstv.at[islot(ci), 0]],
                                  sss[b]).wait()

            @pl.when(ci + 2 < NCHUNK)
            def _():
                pltpu.async_copy(x_hbm.at[srcv.at[islot(ci + 2), 0]],
                                 rows.at[b], sgs[b])

        issue(0, 0)
        issue(1, 1)

        def pair_body(k, carry):
            step(2 * k, 0)
            step(2 * k + 1, 1)
            return carry

        lax.fori_loop(0, NCHUNK // 2, pair_body, 0)
        step(NCHUNK - 1, 0)
        plsc.subcore_barrier()
        pltpu.sync_copy(acc_sh.at[pl.ds(r0, RPW)],
                        out_hbm.at[cid, pl.ds(r0, RPW)])

    return sc_agg


@functools.cache
def _sc_agg_cached():
    return _sc_agg_build()


def _sc_agg_call(x, we, src, dst, zeros_rpw):
    return _sc_agg_cached()(x, we, src, dst, zeros_rpw)


def _norm_act(p_ref, sc_ref, w2_ref, g_ref, brow_ref):
    agg = (p_ref[0] + p_ref[1]) * CN
    y = jnp.dot(agg, w2_ref[...], preferred_element_type=jnp.float32) + sc_ref[...]
    n2 = jnp.dot(y * y, g_ref[...], preferred_element_type=jnp.float32) + 1e-8
    nrm = jnp.sqrt(n2)
    t = nrm + brow_ref[...]
    sp = jnp.maximum(t, 0.0) + jnp.log1p(jnp.exp(-jnp.abs(t))) - LOG2
    return y * (sp / nrm)


def _finish_body(p_ref, sc_ref, w2_ref, g_ref, brow_ref, out_ref):
    out_ref[...] = _norm_act(p_ref, sc_ref, w2_ref, g_ref, brow_ref)


def _mid_body(p_ref, sc_ref, w2_ref, g_ref, brow_ref, attr_ref, w1_ref,
              wsc_ref, x_ref, sc2_ref):
    h = _norm_act(p_ref, sc_ref, w2_ref, g_ref, brow_ref)
    x_ref[...] = jnp.dot(h, w1_ref[...], preferred_element_type=jnp.float32)
    sc2_ref[...] = h * jnp.dot(attr_ref[...], wsc_ref[...],
                               preferred_element_type=jnp.float32)


def _mid(parts, sc, w2, g, brow, attrs, w1n, wscn):
    return pl.pallas_call(
        _mid_body,
        grid=(N // BN,),
        in_specs=[
            pl.BlockSpec((NSC, BN, D), lambda i: (0, i, 0)),
            pl.BlockSpec((BN, D), lambda i: (i, 0)),
            pl.BlockSpec((D, D), lambda i: (0, 0)),
            pl.BlockSpec((D, D), lambda i: (0, 0)),
            pl.BlockSpec((1, D), lambda i: (0, 0)),
            pl.BlockSpec((BN, 16), lambda i: (i, 0)),
            pl.BlockSpec((D, D), lambda i: (0, 0)),
            pl.BlockSpec((16, D), lambda i: (0, 0)),
        ],
        out_specs=[
            pl.BlockSpec((BN, D), lambda i: (i, 0)),
            pl.BlockSpec((BN, D), lambda i: (i, 0)),
        ],
        out_shape=[
            jax.ShapeDtypeStruct((N, D), jnp.float32),
            jax.ShapeDtypeStruct((N, D), jnp.float32),
        ],
    )(parts, sc, w2, g, brow, attrs, w1n, wscn)


def _finish(parts, sc, w2, g, brow):
    return pl.pallas_call(
        _finish_body,
        grid=(N // BN,),
        in_specs=[
            pl.BlockSpec((NSC, BN, D), lambda i: (0, i, 0)),
            pl.BlockSpec((BN, D), lambda i: (i, 0)),
            pl.BlockSpec((D, D), lambda i: (0, 0)),
            pl.BlockSpec((D, D), lambda i: (0, 0)),
            pl.BlockSpec((1, D), lambda i: (0, 0)),
        ],
        out_specs=pl.BlockSpec((BN, D), lambda i: (i, 0)),
        out_shape=jax.ShapeDtypeStruct((N, D), jnp.float32),
    )(parts, sc, w2, g, brow)


def _norm_groups():
    p = jnp.arange(D)[:, None]
    q = jnp.arange(D)[None, :]
    scal = (p == q) & (q < NS)
    vec = (p >= NS) & (q >= NS) & ((p - NS) // 3 == (q - NS) // 3)
    return (scal | vec).astype(jnp.float32)


def kernel(node_features, node_attrs, edge_index, edge_embedding, edge_attrs,
           W1_0, Wfc1_0, Wfc2_0, We_0, W2_0, Wsc_0, bias_0,
           W1_1, Wfc1_1, Wfc2_1, We_1, W2_1, Wsc_1, bias_1):
    src = edge_index[0].astype(jnp.int32).reshape(E // CHUNK, 1, CHUNK)
    dst = edge_index[1].astype(jnp.int32).reshape(E // CHUNK, 1, CHUNK)
    zeros_rpw = jnp.zeros((RPW, D), dtype=jnp.float32)
    g = _norm_groups()
    brow0 = jnp.where(jnp.arange(D) < NS, bias_0[0], bias_0[1])[None, :]
    brow1 = jnp.where(jnp.arange(D) < NS, bias_1[0], bias_1[1])[None, :]

    we1, we2, x1, sc1 = _fused_a(edge_embedding, edge_attrs,
                                 Wfc1_0, Wfc2_0, We_0,
                                 Wfc1_1, Wfc2_1, We_1,
                                 node_features, node_attrs, W1_0, Wsc_0)
    parts1 = _sc_agg_call(x1, we1, src, dst, zeros_rpw)
    x2, sc2 = _mid(parts1, sc1, W2_0, g, brow0, node_attrs, W1_1, Wsc_1)
    parts2 = _sc_agg_call(x2, we2, src, dst, zeros_rpw)
    return _finish(parts2, sc2, W2_1, g, brow1)
